# baseline (speedup 1.0000x reference)
import jax
import jax.numpy as jnp
from jax import lax
from jax.experimental import pallas as pl
from jax.experimental.pallas import tpu as pltpu
from jax.experimental.pallas import tpu_sc as plsc

_B = 16384
_E = 32
_W = _E + 1
_L = 16
_NC = 2
_NS = 16
_NW = _NC * _NS
_BPW = _B // _NW
_CH = 128
_NCH = _BPW // _CH
_G = _BPW // _L


def _mf_body(user_hbm, item_hbm, ue_hbm, ie_hbm, out_hbm,
             uidx, iidx, urows, irows, outv, sem):
    wid = lax.axis_index("s") * _NC + lax.axis_index("c")
    base = wid * _BPW
    row0 = wid * _NCH

    pltpu.sync_copy(user_hbm.at[pl.ds(row0, _NCH)], uidx)
    pltpu.sync_copy(item_hbm.at[pl.ds(row0, _NCH)], iidx)

    copies = []
    for ch in range(_NCH):
        sl = pl.ds(ch * _CH, _CH)
        copies.append(pltpu.async_copy(ue_hbm.at[uidx.at[ch]], urows.at[sl], sem))
        copies.append(pltpu.async_copy(ie_hbm.at[iidx.at[ch]], irows.at[sl], sem))
    for cp in copies:
        cp.wait()

    lane = lax.iota(jnp.int32, _L)
    bcol = jnp.full((_L,), _E, jnp.int32)

    def group(g, carry):
        rows = g * _L + lane
        acc = (plsc.load_gather(urows, [rows, bcol])
               + plsc.load_gather(irows, [rows, bcol]))
        for e in range(_E):
            cols = jnp.full((_L,), e, jnp.int32)
            uu = plsc.load_gather(urows, [rows, cols])
            ii = plsc.load_gather(irows, [rows, cols])
            acc = acc + uu * ii
        outv[pl.ds(g * _L, _L)] = 1.0 / (1.0 + jnp.exp(-acc))
        return carry

    lax.fori_loop(0, _G, group, 0)
    pltpu.sync_copy(outv, out_hbm.at[pl.ds(base, _BPW)])


def kernel(user, item, user_e, item_e, user_b, item_b):
    user2 = user.reshape(_B // _CH, _CH).astype(jnp.int32)
    item2 = item.reshape(_B // _CH, _CH).astype(jnp.int32)
    ue2 = jnp.concatenate([user_e, user_b], axis=1)
    ie2 = jnp.concatenate([item_e, item_b], axis=1)
    mesh = plsc.VectorSubcoreMesh(core_axis_name="c", subcore_axis_name="s")
    f = pl.kernel(
        _mf_body,
        out_type=jax.ShapeDtypeStruct((_B,), jnp.float32),
        mesh=mesh,
        compiler_params=pltpu.CompilerParams(
            needs_layout_passes=False, use_tc_tiling_on_sc=False),
        scratch_types=[
            pltpu.VMEM((_NCH, _CH), jnp.int32),
            pltpu.VMEM((_NCH, _CH), jnp.int32),
            pltpu.VMEM((_BPW, _W), jnp.float32),
            pltpu.VMEM((_BPW, _W), jnp.float32),
            pltpu.VMEM((_BPW,), jnp.float32),
            pltpu.SemaphoreType.DMA,
        ],
    )
    return f(user2, item2, ue2, ie2)


# speedup vs baseline: 2.5538x; 2.5538x over previous
import jax
import jax.numpy as jnp
from jax import lax
from jax.experimental import pallas as pl
from jax.experimental.pallas import tpu as pltpu
from jax.experimental.pallas import tpu_sc as plsc

_B = 16384
_E = 32
_L = 16
_NC = 2
_NS = 16
_NW = _NC * _NS
_BPW = _B // _NW
_CH = 128
_NCH = _BPW // _CH
_G = _BPW // _L


def _mf_body(user_hbm, item_hbm, ue_hbm, ie_hbm, ub_hbm, ib_hbm, out_hbm,
             uidx, iidx, urows, irows, ubv, ibv, outv, sem):
    wid = lax.axis_index("s") * _NC + lax.axis_index("c")
    base = wid * _BPW
    row0 = wid * _NCH

    pltpu.sync_copy(user_hbm.at[pl.ds(row0, _NCH)], uidx)
    pltpu.sync_copy(item_hbm.at[pl.ds(row0, _NCH)], iidx)

    copies = []
    for ch in range(_NCH):
        sl = pl.ds(ch * _CH, _CH)
        copies.append(pltpu.async_copy(ue_hbm.at[uidx.at[ch]], urows.at[sl], sem))
        copies.append(pltpu.async_copy(ie_hbm.at[iidx.at[ch]], irows.at[sl], sem))
        copies.append(pltpu.async_copy(ub_hbm.at[uidx.at[ch]], ubv.at[sl], sem))
        copies.append(pltpu.async_copy(ib_hbm.at[iidx.at[ch]], ibv.at[sl], sem))
    for cp in copies:
        cp.wait()

    lane = lax.iota(jnp.int32, _L)

    def group(g, carry):
        rows = g * _L + lane
        acc = ubv[pl.ds(g * _L, _L)] + ibv[pl.ds(g * _L, _L)]
        for e in range(_E):
            cols = jnp.full((_L,), e, jnp.int32)
            uu = plsc.load_gather(urows, [rows, cols])
            ii = plsc.load_gather(irows, [rows, cols])
            acc = acc + uu * ii
        outv[pl.ds(g * _L, _L)] = 1.0 / (1.0 + jnp.exp(-acc))
        return carry

    lax.fori_loop(0, _G, group, 0)
    pltpu.sync_copy(outv, out_hbm.at[pl.ds(base, _BPW)])


def kernel(user, item, user_e, item_e, user_b, item_b):
    user2 = user.reshape(_B // _CH, _CH).astype(jnp.int32)
    item2 = item.reshape(_B // _CH, _CH).astype(jnp.int32)
    ub = jnp.sum(user_b, axis=1)
    ib = jnp.sum(item_b, axis=1)
    mesh = plsc.VectorSubcoreMesh(core_axis_name="c", subcore_axis_name="s")
    f = pl.kernel(
        _mf_body,
        out_type=jax.ShapeDtypeStruct((_B,), jnp.float32),
        mesh=mesh,
        compiler_params=pltpu.CompilerParams(
            needs_layout_passes=False, use_tc_tiling_on_sc=False),
        scratch_types=[
            pltpu.VMEM((_NCH, _CH), jnp.int32),
            pltpu.VMEM((_NCH, _CH), jnp.int32),
            pltpu.VMEM((_BPW, _E), jnp.float32),
            pltpu.VMEM((_BPW, _E), jnp.float32),
            pltpu.VMEM((_BPW,), jnp.float32),
            pltpu.VMEM((_BPW,), jnp.float32),
            pltpu.VMEM((_BPW,), jnp.float32),
            pltpu.SemaphoreType.DMA,
        ],
    )
    return f(user2, item2, user_e, item_e, ub, ib)


# speedup vs baseline: 2.5555x; 1.0007x over previous
import jax
import jax.numpy as jnp
from jax import lax
from jax.experimental import pallas as pl
from jax.experimental.pallas import tpu as pltpu
from jax.experimental.pallas import tpu_sc as plsc

_B = 16384
_E = 32
_L = 16
_NC = 2
_NS = 16
_NW = _NC * _NS
_BPW = _B // _NW
_CH = 128
_NCH = _BPW // _CH
_G = _BPW // _L


def _mf_body(user_hbm, item_hbm, ue_hbm, ie_hbm, ub_hbm, ib_hbm, out_hbm,
             uidx, iidx, urows, irows, ubv, ibv, outv, sem):
    wid = lax.axis_index("s") * _NC + lax.axis_index("c")
    base = wid * _BPW
    row0 = wid * _NCH

    pltpu.sync_copy(user_hbm.at[pl.ds(row0, _NCH)], uidx)
    pltpu.sync_copy(item_hbm.at[pl.ds(row0, _NCH)], iidx)

    copies = []
    for ch in range(_NCH):
        sl = pl.ds(ch * _CH, _CH)
        copies.append(pltpu.async_copy(ue_hbm.at[uidx.at[ch]], urows.at[sl], sem))
        copies.append(pltpu.async_copy(ie_hbm.at[iidx.at[ch]], irows.at[sl], sem))
        copies.append(pltpu.async_copy(ub_hbm.at[uidx.at[ch]], ubv.at[sl], sem))
        copies.append(pltpu.async_copy(ib_hbm.at[iidx.at[ch]], ibv.at[sl], sem))
    for cp in copies:
        cp.wait()

    lane = lax.iota(jnp.int32, _L)

    def group(g, carry):
        rows = g * _L + lane
        acc = ubv[pl.ds(g * _L, _L)] + ibv[pl.ds(g * _L, _L)]
        for e in range(_E):
            cols = jnp.full((_L,), e, jnp.int32)
            uu = plsc.load_gather(urows, [rows, cols])
            ii = plsc.load_gather(irows, [rows, cols])
            acc = acc + uu * ii
        outv[pl.ds(g * _L, _L)] = 1.0 / (1.0 + jnp.exp(-acc))
        return carry

    lax.fori_loop(0, _G, group, 0)
    pltpu.sync_copy(outv, out_hbm.at[pl.ds(base, _BPW)])


def kernel(user, item, user_e, item_e, user_b, item_b):
    user2 = user.reshape(_B // _CH, _CH).astype(jnp.int32)
    item2 = item.reshape(_B // _CH, _CH).astype(jnp.int32)
    ub = pltpu.with_memory_space_constraint(jnp.sum(user_b, axis=1), pltpu.HBM)
    ib = pltpu.with_memory_space_constraint(jnp.sum(item_b, axis=1), pltpu.HBM)
    mesh = plsc.VectorSubcoreMesh(core_axis_name="c", subcore_axis_name="s")
    f = pl.kernel(
        _mf_body,
        out_type=jax.ShapeDtypeStruct((_B,), jnp.float32),
        mesh=mesh,
        compiler_params=pltpu.CompilerParams(
            needs_layout_passes=False, use_tc_tiling_on_sc=False),
        scratch_types=[
            pltpu.VMEM((_NCH, _CH), jnp.int32),
            pltpu.VMEM((_NCH, _CH), jnp.int32),
            pltpu.VMEM((_BPW, _E), jnp.float32),
            pltpu.VMEM((_BPW, _E), jnp.float32),
            pltpu.VMEM((_BPW,), jnp.float32),
            pltpu.VMEM((_BPW,), jnp.float32),
            pltpu.VMEM((_BPW,), jnp.float32),
            pltpu.SemaphoreType.DMA,
        ],
    )
    return f(user2, item2, user_e, item_e, ub, ib)
